# Initial kernel scaffold; baseline (speedup 1.0000x reference)
import jax
import jax.numpy as jnp
from jax.experimental import pallas as pl


def kernel(x, edge_index, batch, lin0_W, lin0_b, conv_W, conv_b, lstm_Wih, lstm_Whh, lstm_bih, lstm_bhh, lin1_W, lin1_b, lin2_W, lin2_b):
    raise NotImplementedError("write your pallas kernel here")


# speedup vs baseline: 32.2269x; 32.2269x over previous
import functools

import jax
import jax.numpy as jnp
from jax import lax
from jax.experimental import pallas as pl
from jax.experimental.pallas import tpu as pltpu
from jax.experimental.pallas import tpu_sc as plsc

N = 10000
NPAD = 10240
E = 320000
DIN = 128
DIM = 64
G = 64
W = 128
NWIN = 80
EPAD = 32 * NWIN * W
ROWS_PER_TILE = NPAD // 16
HI = lax.Precision.HIGHEST

def _vmesh():
    return plsc.VectorSubcoreMesh(core_axis_name="c", subcore_axis_name="s")


def _sc_degree(dst2d, zero1):
    @functools.partial(
        pl.kernel,
        out_type=jax.ShapeDtypeStruct((2, NPAD), jnp.float32),
        mesh=_vmesh(),
        scratch_types=[
            pltpu.VMEM((NWIN, W), jnp.int32),
            pltpu.VMEM((W,), jnp.float32),
            pltpu.VMEM_SHARED((NPAD,), jnp.float32),
        ],
    )
    def deg_kernel(dst_hbm, zero_hbm, out_hbm, idx_v, ones_v, acc_sh):
        cid = lax.axis_index("c")
        sid = lax.axis_index("s")

        @pl.loop(0, W, step=16)
        def _(i):
            ones_v[pl.ds(i, 16)] = jnp.ones((16,), jnp.float32)

        pltpu.sync_copy(zero_hbm.at[pl.ds(sid * ROWS_PER_TILE, ROWS_PER_TILE)],
                        acc_sh.at[pl.ds(sid * ROWS_PER_TILE, ROWS_PER_TILE)])
        base_row = (cid * 16 + sid) * NWIN
        pltpu.sync_copy(dst_hbm.at[pl.ds(base_row, NWIN)], idx_v)
        plsc.subcore_barrier()

        @pl.loop(0, NWIN)
        def _(j):
            pltpu.sync_copy(ones_v, acc_sh.at[idx_v.at[j]], add=True)

        plsc.subcore_barrier()
        pltpu.sync_copy(acc_sh.at[pl.ds(sid * ROWS_PER_TILE, ROWS_PER_TILE)],
                        out_hbm.at[cid, pl.ds(sid * ROWS_PER_TILE, ROWS_PER_TILE)])

    return deg_kernel(dst2d, zero1)


def _sc_messages(hs, src2d, dst2d, zero2):
    @functools.partial(
        pl.kernel,
        out_type=jax.ShapeDtypeStruct((2, NPAD, DIM), jnp.float32),
        mesh=_vmesh(),
        compiler_params=pltpu.CompilerParams(use_tc_tiling_on_sc=False),
        scratch_types=[
            pltpu.VMEM((NWIN, W), jnp.int32),
            pltpu.VMEM((NWIN, W), jnp.int32),
            pltpu.VMEM((W, DIM), jnp.float32),
            pltpu.VMEM((W, DIM), jnp.float32),
            pltpu.VMEM_SHARED((NPAD, DIM), jnp.float32),
            pltpu.SemaphoreType.DMA,
            pltpu.SemaphoreType.DMA,
        ],
    )
    def msg_kernel(hs_hbm, src_hbm, dst_hbm, zero_hbm, out_hbm,
                   sidx, didx, rows0, rows1, acc_sh, sem0, sem1):
        cid = lax.axis_index("c")
        sid = lax.axis_index("s")
        base_row = (cid * 16 + sid) * NWIN
        pltpu.sync_copy(src_hbm.at[pl.ds(base_row, NWIN)], sidx)
        pltpu.sync_copy(dst_hbm.at[pl.ds(base_row, NWIN)], didx)
        pltpu.sync_copy(zero_hbm.at[pl.ds(sid * ROWS_PER_TILE, ROWS_PER_TILE)],
                        acc_sh.at[pl.ds(sid * ROWS_PER_TILE, ROWS_PER_TILE)])
        plsc.subcore_barrier()

        pltpu.async_copy(hs_hbm.at[sidx.at[0]], rows0, sem0)
        pltpu.async_copy(hs_hbm.at[sidx.at[1]], rows1, sem1)

        @pl.loop(0, NWIN, step=2)
        def _(j0):
            for b in range(2):
                rows = rows0 if b == 0 else rows1
                sem = sem0 if b == 0 else sem1
                j = j0 + b
                pltpu.make_async_copy(hs_hbm.at[sidx.at[j]], rows, sem).wait()
                pltpu.sync_copy(rows, acc_sh.at[didx.at[j]], add=True)

                @pl.when(j + 2 < NWIN)
                def _():
                    pltpu.async_copy(hs_hbm.at[sidx.at[j + 2]], rows, sem)

        plsc.subcore_barrier()
        pltpu.sync_copy(acc_sh.at[pl.ds(sid * ROWS_PER_TILE, ROWS_PER_TILE)],
                        out_hbm.at[cid, pl.ds(sid * ROWS_PER_TILE, ROWS_PER_TILE)])

    return msg_kernel(hs, src2d, dst2d, zero2)


def _tc_pre(x_pad, w0t, b0, wct):
    def body(x_ref, w0_ref, b0_ref, wc_ref, h_ref):
        out0 = jnp.maximum(
            jnp.dot(x_ref[...], w0_ref[...], precision=HI) + b0_ref[...], 0.0)
        h_ref[...] = jnp.dot(out0, wc_ref[...], precision=HI)

    return pl.pallas_call(
        body,
        out_shape=jax.ShapeDtypeStruct((NPAD, DIM), jnp.float32),
    )(x_pad, w0t, b0, wct)


def _tc_scale(degp, h):
    def body(degp_ref, h_ref, hs_ref, dinv_ref):
        d = degp_ref[0] + degp_ref[1] + 1.0
        dinv = lax.rsqrt(d)
        dinv_ref[...] = dinv
        hs_ref[...] = dinv * h_ref[...]

    return pl.pallas_call(
        body,
        out_shape=(jax.ShapeDtypeStruct((NPAD, DIM), jnp.float32),
                   jax.ShapeDtypeStruct((NPAD, 1), jnp.float32)),
    )(degp, h)


def _tc_combine(accp, hs, dinv, convb):
    def body(acc_ref, hs_ref, dinv_ref, cb_ref, node_ref):
        agg = acc_ref[0] + acc_ref[1] + hs_ref[...]
        node_ref[...] = jnp.maximum((dinv_ref[...] * agg + cb_ref[...])[:N],
                                    0.0)

    return pl.pallas_call(
        body,
        out_shape=jax.ShapeDtypeStruct((N, DIM), jnp.float32),
    )(accp, hs, dinv, convb)


def _tc_post(node_in, batch2d, wiht, whht, bsum, l1t, b1, l2t, b2):
    def body(node_ref, b_ref, wih_ref, whh_ref,
             bs_ref, l1_ref, b1_ref, l2_ref, b2_ref, o_ref):
        node = node_ref[...]
        S = (b_ref[...] == lax.broadcasted_iota(jnp.int32, (1, G), 1)
             ).astype(jnp.float32)

        hS = jnp.zeros((G, DIM), jnp.float32)
        cS = jnp.zeros((G, DIM), jnp.float32)
        q_star = jnp.zeros((G, 2 * DIM), jnp.float32)
        for _ in range(3):
            gates = (jnp.dot(q_star, wih_ref[...], precision=HI)
                     + jnp.dot(hS, whh_ref[...], precision=HI) + bs_ref[...])
            ii = jax.nn.sigmoid(gates[:, 0:DIM])
            ff = jax.nn.sigmoid(gates[:, DIM:2 * DIM])
            gg = jnp.tanh(gates[:, 2 * DIM:3 * DIM])
            oo = jax.nn.sigmoid(gates[:, 3 * DIM:4 * DIM])
            cS = ff * cS + ii * gg
            hS = oo * jnp.tanh(cS)
            q = hS
            qb = jnp.dot(S, q, precision=HI)
            e = jnp.sum(node * qb, axis=1, keepdims=True)
            m = jnp.max(jnp.where(S > 0.0, e, -jnp.inf), axis=0,
                        keepdims=True)
            m = jnp.where(jnp.isfinite(m), m, 0.0)
            en = jnp.sum(S * m, axis=1, keepdims=True)
            ee = jnp.exp(e - en)
            esum = lax.dot_general(ee, S, (((0,), (0,)), ((), ())),
                                   precision=HI)
            den = jnp.sum(S * esum, axis=1, keepdims=True) + 1e-16
            aa = ee / den
            r = lax.dot_general(S, aa * node, (((0,), (0,)), ((), ())),
                                precision=HI)
            q_star = jnp.concatenate([q, r], axis=1)

        out2 = jnp.maximum(
            jnp.dot(q_star, l1_ref[...], precision=HI) + b1_ref[...], 0.0)
        o_ref[...] = jnp.dot(out2, l2_ref[...], precision=HI) + b2_ref[...]

    return pl.pallas_call(
        body,
        out_shape=jax.ShapeDtypeStruct((G, 1), jnp.float32),
    )(node_in, batch2d, wiht, whht, bsum, l1t, b1, l2t, b2)


def kernel(x, edge_index, batch, lin0_W, lin0_b, conv_W, conv_b,
           lstm_Wih, lstm_Whh, lstm_bih, lstm_bhh,
           lin1_W, lin1_b, lin2_W, lin2_b):
    src = edge_index[0]
    dst = edge_index[1]

    pad = EPAD - E
    pad_idx = (jnp.arange(pad, dtype=jnp.int32) % (NPAD - N)) + N
    src2d = jnp.concatenate([src, pad_idx]).reshape(EPAD // W, W)
    dst2d = jnp.concatenate([dst, pad_idx]).reshape(EPAD // W, W)

    x_pad = jnp.concatenate(
        [x, jnp.zeros((NPAD - N, DIN), jnp.float32)], axis=0)
    zero1 = jnp.zeros((NPAD,), jnp.float32)
    zero2 = jnp.zeros((NPAD, DIM), jnp.float32)

    degp = _sc_degree(dst2d, zero1)
    h = _tc_pre(x_pad, lin0_W.T, lin0_b.reshape(1, DIM), conv_W.T)
    hs, dinv = _tc_scale(degp.reshape(2, NPAD, 1), h)
    accp = _sc_messages(hs, src2d, dst2d, zero2)

    node = _tc_combine(accp, hs, dinv, conv_b.reshape(1, DIM))
    res = _tc_post(node, batch.reshape(N, 1),
                   lstm_Wih.T, lstm_Whh.T,
                   (lstm_bih + lstm_bhh).reshape(1, 4 * DIM),
                   lin1_W.T, lin1_b.reshape(1, DIM),
                   lin2_W.T, lin2_b.reshape(1, 1))
    return res.reshape(-1)
